# baseline (speedup 1.0000x reference)
import functools

import jax
import jax.numpy as jnp
from jax import lax
from jax.experimental import pallas as pl
from jax.experimental.pallas import tpu as pltpu
from jax.experimental.pallas import tpu_sc as plsc

_K = 3
_TILE = 256
_LANES = 128


def _topk_tc_body(sess_ref, w1_ref, b1_ref, w2_ref, b2_ref,
                  w_ref, wp_ref, idxp_ref, proj_ref):
    i = pl.program_id(0)

    @pl.when(i == 0)
    def _():
        h = jnp.maximum(
            jnp.dot(sess_ref[...], w1_ref[...],
                    preferred_element_type=jnp.float32) + b1_ref[...], 0.0)
        proj_ref[...] = jnp.dot(h, w2_ref[...],
                                preferred_element_type=jnp.float32) + b2_ref[...]

    tile = proj_ref[pl.ds(i * _TILE, _TILE), :]
    full = proj_ref[...]
    b = full.shape[0]
    sim = lax.dot_general(tile, full, (((1,), (1,)), ((), ())),
                          preferred_element_type=jnp.float32)

    neg = jnp.float32(-3e38)
    lane_iota = lax.broadcasted_iota(jnp.int32, (_TILE, _LANES), 1)
    t1 = jnp.full((_TILE, _LANES), neg, jnp.float32)
    t2 = t1
    t3 = t1
    i1 = jnp.full((_TILE, _LANES), b, jnp.int32)
    i2 = i1
    i3 = i1
    for k in range(b // _LANES):
        v = sim[:, k * _LANES:(k + 1) * _LANES]
        iv = lane_iota + (k * _LANES)
        c1 = v > t1
        nt1 = jnp.maximum(t1, v)
        dv = jnp.minimum(t1, v)
        ni1 = jnp.where(c1, iv, i1)
        di = jnp.where(c1, i1, iv)
        c2 = dv > t2
        nt2 = jnp.maximum(t2, dv)
        dv2 = jnp.minimum(t2, dv)
        ni2 = jnp.where(c2, di, i2)
        di2 = jnp.where(c2, i2, di)
        c3 = dv2 > t3
        t3 = jnp.maximum(t3, dv2)
        i3 = jnp.where(c3, di2, i3)
        t1, t2, i1, i2 = nt1, nt2, ni1, ni2

    cand = jnp.concatenate([t1, t2, t3], axis=1)
    cidx = jnp.concatenate([i1, i2, i3], axis=1)
    vals, idxs = [], []
    for _ in range(_K):
        m = jnp.max(cand, axis=1)
        j = jnp.min(jnp.where(cand == m[:, None], cidx, b), axis=1)
        vals.append(m)
        idxs.append(j)
        cand = jnp.where(cidx == j[:, None], neg, cand)

    e0 = jnp.ones_like(vals[0])
    e1 = jnp.exp(vals[1] - vals[0])
    e2 = jnp.exp(vals[2] - vals[0])
    s = e0 + e1 + e2
    ws = [e0 / s, e1 / s, e2 / s]
    w_ref[...] = jnp.concatenate(
        [ws[0][:, None], ws[1][:, None], ws[2][:, None]], axis=1)
    wp_ref[...] = jnp.concatenate(
        [ws[0][None, :], ws[1][None, :], ws[2][None, :]], axis=0)
    idxp_ref[...] = jnp.concatenate(
        [idxs[0][None, :], idxs[1][None, :], idxs[2][None, :]], axis=0)


def _topk_tc(sess_emb, W1, b1, W2, b2):
    b, d = sess_emb.shape
    grid = b // _TILE
    full_spec = lambda arr: pl.BlockSpec(arr.shape, lambda i: (0,) * arr.ndim)
    w_out = jax.ShapeDtypeStruct((b, _K), jnp.float32)
    wp_out = jax.ShapeDtypeStruct((_K, b), jnp.float32)
    idxp_out = jax.ShapeDtypeStruct((_K, b), jnp.int32)
    return pl.pallas_call(
        _topk_tc_body,
        grid=(grid,),
        in_specs=[full_spec(sess_emb), full_spec(W1), full_spec(b1),
                  full_spec(W2), full_spec(b2)],
        out_specs=[pl.BlockSpec((_TILE, _K), lambda i: (i, 0)),
                   pl.BlockSpec((_K, _TILE), lambda i: (0, i)),
                   pl.BlockSpec((_K, _TILE), lambda i: (0, i))],
        out_shape=[w_out, wp_out, idxp_out],
        scratch_shapes=[pltpu.VMEM((b, d), jnp.float32)],
    )(sess_emb, W1, b1, W2, b2)


def _gather_combine_sc(sess_emb, idx_p, w_p):
    b, d = sess_emb.shape
    info = plsc.get_sparse_core_info()
    nc, ns = info.num_cores, info.num_subcores
    nw = nc * ns
    rows_w = b // nw
    mesh = plsc.VectorSubcoreMesh(core_axis_name="c", subcore_axis_name="s")

    @functools.partial(
        pl.kernel,
        mesh=mesh,
        compiler_params=pltpu.CompilerParams(use_tc_tiling_on_sc=False),
        out_type=[jax.ShapeDtypeStruct((b, _K, d), jnp.float32),
                  jax.ShapeDtypeStruct((b, d), jnp.float32)],
        scratch_types=[
            [pltpu.VMEM((rows_w,), jnp.int32) for _ in range(_K)],
            [pltpu.VMEM((rows_w, d), jnp.float32) for _ in range(_K)],
            [pltpu.VMEM((rows_w + 16,), jnp.float32) for _ in range(_K)],
            pltpu.VMEM((rows_w, d), jnp.float32),
            pltpu.SemaphoreType.DMA,
        ],
    )
    def sc_kernel(emb_hbm, idx_hbm, w_hbm, topk_out, nb_out,
                  idx_vs, rows_vs, w_vs, acc_v, sem):
        wid = lax.axis_index("s") * nc + lax.axis_index("c")
        base_r = wid * rows_w
        for k in range(_K):
            pltpu.sync_copy(idx_hbm.at[k, pl.ds(base_r, rows_w)], idx_vs[k])
            pltpu.sync_copy(w_hbm.at[k, pl.ds(base_r, rows_w)],
                            w_vs[k].at[pl.ds(0, rows_w)])
        copies = [pltpu.async_copy(emb_hbm.at[idx_vs[k]], rows_vs[k], sem)
                  for k in range(_K)]
        for c in copies:
            c.wait()
        for k in range(_K):
            pltpu.sync_copy(rows_vs[k], topk_out.at[pl.ds(base_r, rows_w), k])

        def body(r, carry):
            acc = (rows_vs[0][r, :] * w_vs[0][pl.ds(r, 16)][0]
                   + rows_vs[1][r, :] * w_vs[1][pl.ds(r, 16)][0]
                   + rows_vs[2][r, :] * w_vs[2][pl.ds(r, 16)][0])
            acc_v[r, :] = acc
            return carry

        lax.fori_loop(0, rows_w, body, 0)
        pltpu.sync_copy(acc_v, nb_out.at[pl.ds(base_r, rows_w)])

    return sc_kernel(sess_emb, idx_p, w_p)


def kernel(sess_emb, pool_emb, W1, b1, W2, b2):
    del pool_emb
    w, w_p, idx_p = _topk_tc(sess_emb, W1, b1.reshape(1, -1), W2,
                             b2.reshape(1, -1))
    sess_topk, neighbor_sess = _gather_combine_sc(sess_emb, idx_p, w_p)
    return (sess_topk, neighbor_sess, w)


# speedup vs baseline: 1.0270x; 1.0270x over previous
import functools

import jax
import jax.numpy as jnp
from jax import lax
from jax.experimental import pallas as pl
from jax.experimental.pallas import tpu as pltpu
from jax.experimental.pallas import tpu_sc as plsc

_K = 3
_TILE = 256
_LANES = 128


def _topk_tc_body(sess_ref, w1_ref, b1_ref, w2_ref, b2_ref,
                  w_ref, wp_ref, idxp_ref, proj_ref):
    i = pl.program_id(0)

    @pl.when(i == 0)
    def _():
        h = jnp.maximum(
            jnp.dot(sess_ref[...], w1_ref[...],
                    preferred_element_type=jnp.float32) + b1_ref[...], 0.0)
        proj_ref[...] = jnp.dot(h, w2_ref[...],
                                preferred_element_type=jnp.float32) + b2_ref[...]

    tile = proj_ref[pl.ds(i * _TILE, _TILE), :]
    full = proj_ref[...]
    b = full.shape[0]
    sim = lax.dot_general(tile, full, (((1,), (1,)), ((), ())),
                          preferred_element_type=jnp.float32)

    neg = jnp.float32(-3e38)
    lane_iota = lax.broadcasted_iota(jnp.int32, (_TILE, _LANES), 1)
    t1 = jnp.full((_TILE, _LANES), neg, jnp.float32)
    t2 = t1
    t3 = t1
    i1 = jnp.full((_TILE, _LANES), b, jnp.int32)
    i2 = i1
    i3 = i1
    for k in range(b // _LANES):
        v = sim[:, k * _LANES:(k + 1) * _LANES]
        iv = lane_iota + (k * _LANES)
        c1 = v > t1
        nt1 = jnp.maximum(t1, v)
        dv = jnp.minimum(t1, v)
        ni1 = jnp.where(c1, iv, i1)
        di = jnp.where(c1, i1, iv)
        c2 = dv > t2
        nt2 = jnp.maximum(t2, dv)
        dv2 = jnp.minimum(t2, dv)
        ni2 = jnp.where(c2, di, i2)
        di2 = jnp.where(c2, i2, di)
        c3 = dv2 > t3
        t3 = jnp.maximum(t3, dv2)
        i3 = jnp.where(c3, di2, i3)
        t1, t2, i1, i2 = nt1, nt2, ni1, ni2

    cand = jnp.concatenate([t1, t2, t3], axis=1)
    cidx = jnp.concatenate([i1, i2, i3], axis=1)
    vals, idxs = [], []
    for _ in range(_K):
        m = jnp.max(cand, axis=1)
        j = jnp.min(jnp.where(cand == m[:, None], cidx, b), axis=1)
        vals.append(m)
        idxs.append(j)
        cand = jnp.where(cidx == j[:, None], neg, cand)

    e0 = jnp.ones_like(vals[0])
    e1 = jnp.exp(vals[1] - vals[0])
    e2 = jnp.exp(vals[2] - vals[0])
    s = e0 + e1 + e2
    ws = [e0 / s, e1 / s, e2 / s]
    w_ref[...] = jnp.concatenate(
        [ws[0][:, None], ws[1][:, None], ws[2][:, None]], axis=1)
    wp_ref[...] = jnp.concatenate(
        [ws[0][None, :], ws[1][None, :], ws[2][None, :],
         jnp.zeros((5, _TILE), jnp.float32)], axis=0)
    idxp_ref[...] = jnp.concatenate(
        [idxs[0][None, :], idxs[1][None, :], idxs[2][None, :],
         jnp.zeros((5, _TILE), jnp.int32)], axis=0)


def _topk_tc(sess_emb, W1, b1, W2, b2):
    b, d = sess_emb.shape
    grid = b // _TILE
    full_spec = lambda arr: pl.BlockSpec(arr.shape, lambda i: (0,) * arr.ndim)
    w_out = jax.ShapeDtypeStruct((b, _K), jnp.float32)
    wp_out = jax.ShapeDtypeStruct((8, b), jnp.float32)
    idxp_out = jax.ShapeDtypeStruct((8, b), jnp.int32)
    return pl.pallas_call(
        _topk_tc_body,
        grid=(grid,),
        in_specs=[full_spec(sess_emb), full_spec(W1), full_spec(b1),
                  full_spec(W2), full_spec(b2)],
        out_specs=[pl.BlockSpec((_TILE, _K), lambda i: (i, 0)),
                   pl.BlockSpec((8, _TILE), lambda i: (0, i)),
                   pl.BlockSpec((8, _TILE), lambda i: (0, i))],
        out_shape=[w_out, wp_out, idxp_out],
        scratch_shapes=[pltpu.VMEM((b, d), jnp.float32)],
    )(sess_emb, W1, b1, W2, b2)


def _gather_combine_sc(sess_emb, idx_p, w_p):
    b, d = sess_emb.shape
    info = plsc.get_sparse_core_info()
    nc, ns = info.num_cores, info.num_subcores
    nw = nc * ns
    rows_w = b // nw
    mesh = plsc.VectorSubcoreMesh(core_axis_name="c", subcore_axis_name="s")

    @functools.partial(
        pl.kernel,
        mesh=mesh,
        compiler_params=pltpu.CompilerParams(use_tc_tiling_on_sc=False),
        out_type=[jax.ShapeDtypeStruct((b, _K, d), jnp.float32),
                  jax.ShapeDtypeStruct((b, d), jnp.float32)],
        scratch_types=[
            [pltpu.VMEM((rows_w,), jnp.int32) for _ in range(_K)],
            [pltpu.VMEM((rows_w, d), jnp.float32) for _ in range(_K)],
            [pltpu.VMEM((rows_w + 16,), jnp.float32) for _ in range(_K)],
            pltpu.VMEM((rows_w, d), jnp.float32),
            pltpu.SemaphoreType.DMA,
        ],
    )
    def sc_kernel(emb_hbm, idx_hbm, w_hbm, topk_out, nb_out,
                  idx_vs, rows_vs, w_vs, acc_v, sem):
        wid = lax.axis_index("s") * nc + lax.axis_index("c")
        base_r = wid * rows_w
        idx_cps = [pltpu.async_copy(idx_hbm.at[k, pl.ds(base_r, rows_w)],
                                    idx_vs[k], sem) for k in range(_K)]
        w_cps = [pltpu.async_copy(w_hbm.at[k, pl.ds(base_r, rows_w)],
                                  w_vs[k].at[pl.ds(0, rows_w)], sem)
                 for k in range(_K)]
        for c in idx_cps:
            c.wait()
        row_cps = [pltpu.async_copy(emb_hbm.at[idx_vs[k]], rows_vs[k], sem)
                   for k in range(_K)]
        for c in w_cps:
            c.wait()
        for c in row_cps:
            c.wait()
        out_cps = [pltpu.async_copy(rows_vs[k],
                                    topk_out.at[pl.ds(base_r, rows_w), k], sem)
                   for k in range(_K)]

        def body(r, carry):
            acc = (rows_vs[0][r, :] * w_vs[0][pl.ds(r, 16)][0]
                   + rows_vs[1][r, :] * w_vs[1][pl.ds(r, 16)][0]
                   + rows_vs[2][r, :] * w_vs[2][pl.ds(r, 16)][0])
            acc_v[r, :] = acc
            return carry

        lax.fori_loop(0, rows_w, body, 0)
        pltpu.sync_copy(acc_v, nb_out.at[pl.ds(base_r, rows_w)])
        for c in out_cps:
            c.wait()

    return sc_kernel(sess_emb, idx_p, w_p)


def kernel(sess_emb, pool_emb, W1, b1, W2, b2):
    del pool_emb
    w, w_p, idx_p = _topk_tc(sess_emb, W1, b1.reshape(1, -1), W2,
                             b2.reshape(1, -1))
    sess_topk, neighbor_sess = _gather_combine_sc(sess_emb, idx_p, w_p)
    return (sess_topk, neighbor_sess, w)


# speedup vs baseline: 1.0869x; 1.0584x over previous
import functools

import jax
import jax.numpy as jnp
from jax import lax
from jax.experimental import pallas as pl
from jax.experimental.pallas import tpu as pltpu
from jax.experimental.pallas import tpu_sc as plsc

_K = 3
_TILE = 256
_LANES = 128


def _topk_tc_body(sess_ref, w1_ref, b1_ref, w2_ref, b2_ref,
                  w_ref, wp_ref, idxp_ref, proj_ref, *, row0):
    i = pl.program_id(0)

    @pl.when(i == 0)
    def _():
        h = jnp.maximum(
            jnp.dot(sess_ref[...], w1_ref[...],
                    preferred_element_type=jnp.float32) + b1_ref[...], 0.0)
        proj_ref[...] = jnp.dot(h, w2_ref[...],
                                preferred_element_type=jnp.float32) + b2_ref[...]

    tile = proj_ref[pl.ds(row0 + i * _TILE, _TILE), :]
    full = proj_ref[...]
    b = full.shape[0]
    sim = lax.dot_general(tile, full, (((1,), (1,)), ((), ())),
                          preferred_element_type=jnp.float32)

    neg = jnp.float32(-3e38)
    lane_iota = lax.broadcasted_iota(jnp.int32, (_TILE, _LANES), 1)
    t1 = jnp.full((_TILE, _LANES), neg, jnp.float32)
    t2 = t1
    t3 = t1
    i1 = jnp.full((_TILE, _LANES), b, jnp.int32)
    i2 = i1
    i3 = i1
    for k in range(b // _LANES):
        v = sim[:, k * _LANES:(k + 1) * _LANES]
        iv = lane_iota + (k * _LANES)
        c1 = v > t1
        nt1 = jnp.maximum(t1, v)
        dv = jnp.minimum(t1, v)
        ni1 = jnp.where(c1, iv, i1)
        di = jnp.where(c1, i1, iv)
        c2 = dv > t2
        nt2 = jnp.maximum(t2, dv)
        dv2 = jnp.minimum(t2, dv)
        ni2 = jnp.where(c2, di, i2)
        di2 = jnp.where(c2, i2, di)
        c3 = dv2 > t3
        t3 = jnp.maximum(t3, dv2)
        i3 = jnp.where(c3, di2, i3)
        t1, t2, i1, i2 = nt1, nt2, ni1, ni2

    cand = jnp.concatenate([t1, t2, t3], axis=1)
    cidx = jnp.concatenate([i1, i2, i3], axis=1)
    vals, idxs = [], []
    for _ in range(_K):
        m = jnp.max(cand, axis=1)
        j = jnp.min(jnp.where(cand == m[:, None], cidx, b), axis=1)
        vals.append(m)
        idxs.append(j)
        cand = jnp.where(cidx == j[:, None], neg, cand)

    e0 = jnp.ones_like(vals[0])
    e1 = jnp.exp(vals[1] - vals[0])
    e2 = jnp.exp(vals[2] - vals[0])
    s = e0 + e1 + e2
    ws = [e0 / s, e1 / s, e2 / s]
    w_ref[...] = jnp.concatenate(
        [ws[0][:, None], ws[1][:, None], ws[2][:, None]], axis=1)
    wp_ref[...] = jnp.concatenate(
        [ws[0][None, :], ws[1][None, :], ws[2][None, :],
         jnp.zeros((5, _TILE), jnp.float32)], axis=0)
    idxp_ref[...] = jnp.concatenate(
        [idxs[0][None, :], idxs[1][None, :], idxs[2][None, :],
         jnp.zeros((5, _TILE), jnp.int32)], axis=0)


def _topk_tc(sess_emb, W1, b1, W2, b2, row0, nrows):
    b, d = sess_emb.shape
    grid = nrows // _TILE
    full_spec = lambda arr: pl.BlockSpec(arr.shape, lambda i: (0,) * arr.ndim)
    w_out = jax.ShapeDtypeStruct((nrows, _K), jnp.float32)
    wp_out = jax.ShapeDtypeStruct((8, nrows), jnp.float32)
    idxp_out = jax.ShapeDtypeStruct((8, nrows), jnp.int32)
    return pl.pallas_call(
        functools.partial(_topk_tc_body, row0=row0),
        grid=(grid,),
        in_specs=[full_spec(sess_emb), full_spec(W1), full_spec(b1),
                  full_spec(W2), full_spec(b2)],
        out_specs=[pl.BlockSpec((_TILE, _K), lambda i: (i, 0)),
                   pl.BlockSpec((8, _TILE), lambda i: (0, i)),
                   pl.BlockSpec((8, _TILE), lambda i: (0, i))],
        out_shape=[w_out, wp_out, idxp_out],
        scratch_shapes=[pltpu.VMEM((b, d), jnp.float32)],
    )(sess_emb, W1, b1, W2, b2)


def _gather_combine_sc(sess_emb, idx_p, w_p):
    b, d = sess_emb.shape
    nrows = idx_p.shape[1]
    info = plsc.get_sparse_core_info()
    nc, ns = info.num_cores, info.num_subcores
    nw = nc * ns
    rows_w = nrows // nw
    mesh = plsc.VectorSubcoreMesh(core_axis_name="c", subcore_axis_name="s")

    @functools.partial(
        pl.kernel,
        mesh=mesh,
        compiler_params=pltpu.CompilerParams(use_tc_tiling_on_sc=False),
        out_type=[jax.ShapeDtypeStruct((nrows, _K, d), jnp.float32),
                  jax.ShapeDtypeStruct((nrows, d), jnp.float32)],
        scratch_types=[
            [pltpu.VMEM((rows_w,), jnp.int32) for _ in range(_K)],
            [pltpu.VMEM((rows_w, d), jnp.float32) for _ in range(_K)],
            [pltpu.VMEM((rows_w + 16,), jnp.float32) for _ in range(_K)],
            pltpu.VMEM((rows_w, d), jnp.float32),
            pltpu.SemaphoreType.DMA,
        ],
    )
    def sc_kernel(emb_hbm, idx_hbm, w_hbm, topk_out, nb_out,
                  idx_vs, rows_vs, w_vs, acc_v, sem):
        wid = lax.axis_index("s") * nc + lax.axis_index("c")
        base_r = wid * rows_w
        idx_cps = [pltpu.async_copy(idx_hbm.at[k, pl.ds(base_r, rows_w)],
                                    idx_vs[k], sem) for k in range(_K)]
        w_cps = [pltpu.async_copy(w_hbm.at[k, pl.ds(base_r, rows_w)],
                                  w_vs[k].at[pl.ds(0, rows_w)], sem)
                 for k in range(_K)]
        for c in idx_cps:
            c.wait()
        row_cps = [pltpu.async_copy(emb_hbm.at[idx_vs[k]], rows_vs[k], sem)
                   for k in range(_K)]
        for c in w_cps:
            c.wait()
        for c in row_cps:
            c.wait()
        out_cps = [pltpu.async_copy(rows_vs[k],
                                    topk_out.at[pl.ds(base_r, rows_w), k], sem)
                   for k in range(_K)]

        def body(r, carry):
            acc = (rows_vs[0][r, :] * w_vs[0][pl.ds(r, 16)][0]
                   + rows_vs[1][r, :] * w_vs[1][pl.ds(r, 16)][0]
                   + rows_vs[2][r, :] * w_vs[2][pl.ds(r, 16)][0])
            acc_v[r, :] = acc
            return carry

        lax.fori_loop(0, rows_w, body, 0)
        pltpu.sync_copy(acc_v, nb_out.at[pl.ds(base_r, rows_w)])
        for c in out_cps:
            c.wait()

    return sc_kernel(sess_emb, idx_p, w_p)


def kernel(sess_emb, pool_emb, W1, b1, W2, b2):
    del pool_emb
    b, d = sess_emb.shape
    half = b // 2
    b1r = b1.reshape(1, -1)
    b2r = b2.reshape(1, -1)
    w0, wp0, ip0 = _topk_tc(sess_emb, W1, b1r, W2, b2r, 0, half)
    st0, nb0 = _gather_combine_sc(sess_emb, ip0, wp0)
    w1, wp1, ip1 = _topk_tc(sess_emb, W1, b1r, W2, b2r, half, half)
    st1, nb1 = _gather_combine_sc(sess_emb, ip1, wp1)
    sess_topk = jnp.concatenate([st0, st1], axis=0)
    neighbor_sess = jnp.concatenate([nb0, nb1], axis=0)
    w = jnp.concatenate([w0, w1], axis=0)
    return (sess_topk, neighbor_sess, w)


# speedup vs baseline: 1.1895x; 1.0943x over previous
import functools

import jax
import jax.numpy as jnp
from jax import lax
from jax.experimental import pallas as pl
from jax.experimental.pallas import tpu as pltpu
from jax.experimental.pallas import tpu_sc as plsc

_K = 3
_TILE = 1024
_LANES = 128


def _topk_tc_body(sess_ref, w1_ref, b1_ref, w2_ref, b2_ref,
                  w_ref, wp_ref, idxp_ref, proj_ref, *, row0):
    i = pl.program_id(0)

    @pl.when(i == 0)
    def _():
        h = jnp.maximum(
            jnp.dot(sess_ref[...], w1_ref[...],
                    preferred_element_type=jnp.float32) + b1_ref[...], 0.0)
        proj_ref[...] = jnp.dot(h, w2_ref[...],
                                preferred_element_type=jnp.float32) + b2_ref[...]

    tile = proj_ref[pl.ds(row0 + i * _TILE, _TILE), :]
    full = proj_ref[...]
    b = full.shape[0]
    sim = lax.dot_general(tile, full, (((1,), (1,)), ((), ())),
                          preferred_element_type=jnp.float32)

    neg = jnp.float32(-3e38)
    lane_iota = lax.broadcasted_iota(jnp.int32, (_TILE, _LANES), 1)
    t1 = jnp.full((_TILE, _LANES), neg, jnp.float32)
    t2 = t1
    t3 = t1
    i1 = jnp.full((_TILE, _LANES), b, jnp.int32)
    i2 = i1
    i3 = i1
    for k in range(b // _LANES):
        v = sim[:, k * _LANES:(k + 1) * _LANES]
        iv = lane_iota + (k * _LANES)
        c1 = v > t1
        nt1 = jnp.maximum(t1, v)
        dv = jnp.minimum(t1, v)
        ni1 = jnp.where(c1, iv, i1)
        di = jnp.where(c1, i1, iv)
        c2 = dv > t2
        nt2 = jnp.maximum(t2, dv)
        dv2 = jnp.minimum(t2, dv)
        ni2 = jnp.where(c2, di, i2)
        di2 = jnp.where(c2, i2, di)
        c3 = dv2 > t3
        t3 = jnp.maximum(t3, dv2)
        i3 = jnp.where(c3, di2, i3)
        t1, t2, i1, i2 = nt1, nt2, ni1, ni2

    cand = jnp.concatenate([t1, t2, t3], axis=1)
    cidx = jnp.concatenate([i1, i2, i3], axis=1)
    vals, idxs = [], []
    for _ in range(_K):
        m = jnp.max(cand, axis=1)
        j = jnp.min(jnp.where(cand == m[:, None], cidx, b), axis=1)
        vals.append(m)
        idxs.append(j)
        cand = jnp.where(cidx == j[:, None], neg, cand)

    e0 = jnp.ones_like(vals[0])
    e1 = jnp.exp(vals[1] - vals[0])
    e2 = jnp.exp(vals[2] - vals[0])
    s = e0 + e1 + e2
    ws = [e0 / s, e1 / s, e2 / s]
    w_ref[...] = jnp.concatenate(
        [ws[0][:, None], ws[1][:, None], ws[2][:, None]], axis=1)
    wp_ref[...] = jnp.concatenate(
        [ws[0][None, :], ws[1][None, :], ws[2][None, :],
         jnp.zeros((5, _TILE), jnp.float32)], axis=0)
    idxp_ref[...] = jnp.concatenate(
        [idxs[0][None, :], idxs[1][None, :], idxs[2][None, :],
         jnp.zeros((5, _TILE), jnp.int32)], axis=0)


def _topk_tc(sess_emb, W1, b1, W2, b2, row0, nrows):
    b, d = sess_emb.shape
    grid = nrows // _TILE
    full_spec = lambda arr: pl.BlockSpec(arr.shape, lambda i: (0,) * arr.ndim)
    w_out = jax.ShapeDtypeStruct((nrows, _K), jnp.float32)
    wp_out = jax.ShapeDtypeStruct((8, nrows), jnp.float32)
    idxp_out = jax.ShapeDtypeStruct((8, nrows), jnp.int32)
    return pl.pallas_call(
        functools.partial(_topk_tc_body, row0=row0),
        grid=(grid,),
        in_specs=[full_spec(sess_emb), full_spec(W1), full_spec(b1),
                  full_spec(W2), full_spec(b2)],
        out_specs=[pl.BlockSpec((_TILE, _K), lambda i: (i, 0)),
                   pl.BlockSpec((8, _TILE), lambda i: (0, i)),
                   pl.BlockSpec((8, _TILE), lambda i: (0, i))],
        out_shape=[w_out, wp_out, idxp_out],
        scratch_shapes=[pltpu.VMEM((b, d), jnp.float32)],
    )(sess_emb, W1, b1, W2, b2)


def _gather_combine_sc(sess_emb, idx_p, w_p):
    b, d = sess_emb.shape
    nrows = idx_p.shape[1]
    info = plsc.get_sparse_core_info()
    nc, ns = info.num_cores, info.num_subcores
    nw = nc * ns
    rows_w = nrows // nw
    mesh = plsc.VectorSubcoreMesh(core_axis_name="c", subcore_axis_name="s")

    @functools.partial(
        pl.kernel,
        mesh=mesh,
        compiler_params=pltpu.CompilerParams(use_tc_tiling_on_sc=False),
        out_type=[jax.ShapeDtypeStruct((nrows, _K, d), jnp.float32),
                  jax.ShapeDtypeStruct((nrows, d), jnp.float32)],
        scratch_types=[
            [pltpu.VMEM((rows_w,), jnp.int32) for _ in range(_K)],
            [pltpu.VMEM((rows_w, d), jnp.float32) for _ in range(_K)],
            [pltpu.VMEM((rows_w + 16,), jnp.float32) for _ in range(_K)],
            pltpu.VMEM((rows_w, d), jnp.float32),
            pltpu.SemaphoreType.DMA,
        ],
    )
    def sc_kernel(emb_hbm, idx_hbm, w_hbm, topk_out, nb_out,
                  idx_vs, rows_vs, w_vs, acc_v, sem):
        wid = lax.axis_index("s") * nc + lax.axis_index("c")
        base_r = wid * rows_w
        idx_cps = [pltpu.async_copy(idx_hbm.at[k, pl.ds(base_r, rows_w)],
                                    idx_vs[k], sem) for k in range(_K)]
        w_cps = [pltpu.async_copy(w_hbm.at[k, pl.ds(base_r, rows_w)],
                                  w_vs[k].at[pl.ds(0, rows_w)], sem)
                 for k in range(_K)]
        for c in idx_cps:
            c.wait()
        row_cps = [pltpu.async_copy(emb_hbm.at[idx_vs[k]], rows_vs[k], sem)
                   for k in range(_K)]
        for c in w_cps:
            c.wait()
        for c in row_cps:
            c.wait()
        out_cps = [pltpu.async_copy(rows_vs[k],
                                    topk_out.at[pl.ds(base_r, rows_w), k], sem)
                   for k in range(_K)]

        def body(r, carry):
            acc = (rows_vs[0][r, :] * w_vs[0][pl.ds(r, 16)][0]
                   + rows_vs[1][r, :] * w_vs[1][pl.ds(r, 16)][0]
                   + rows_vs[2][r, :] * w_vs[2][pl.ds(r, 16)][0])
            acc_v[r, :] = acc
            return carry

        lax.fori_loop(0, rows_w, body, 0)
        pltpu.sync_copy(acc_v, nb_out.at[pl.ds(base_r, rows_w)])
        for c in out_cps:
            c.wait()

    return sc_kernel(sess_emb, idx_p, w_p)


def kernel(sess_emb, pool_emb, W1, b1, W2, b2):
    del pool_emb
    b, d = sess_emb.shape
    half = b // 2
    b1r = b1.reshape(1, -1)
    b2r = b2.reshape(1, -1)
    w0, wp0, ip0 = _topk_tc(sess_emb, W1, b1r, W2, b2r, 0, half)
    st0, nb0 = _gather_combine_sc(sess_emb, ip0, wp0)
    w1, wp1, ip1 = _topk_tc(sess_emb, W1, b1r, W2, b2r, half, half)
    st1, nb1 = _gather_combine_sc(sess_emb, ip1, wp1)
    sess_topk = jnp.concatenate([st0, st1], axis=0)
    neighbor_sess = jnp.concatenate([nb0, nb1], axis=0)
    w = jnp.concatenate([w0, w1], axis=0)
    return (sess_topk, neighbor_sess, w)


# speedup vs baseline: 1.5214x; 1.2791x over previous
import functools

import jax
import jax.numpy as jnp
from jax import lax
from jax.experimental import pallas as pl
from jax.experimental.pallas import tpu as pltpu
from jax.experimental.pallas import tpu_sc as plsc

_K = 3
_TILE = 1024
_LANES = 128


def _topk_tc_body(sess_ref, w1_ref, b1_ref, w2_ref, b2_ref,
                  w_ref, wp_ref, idxp_ref, proj_ref, *, row0):
    i = pl.program_id(0)

    @pl.when(i == 0)
    def _():
        h = jnp.maximum(
            jnp.dot(sess_ref[...], w1_ref[...],
                    preferred_element_type=jnp.float32) + b1_ref[...], 0.0)
        proj_ref[...] = jnp.dot(h, w2_ref[...],
                                preferred_element_type=jnp.float32) + b2_ref[...]

    tile = proj_ref[pl.ds(row0 + i * _TILE, _TILE), :]
    full = proj_ref[...]
    b = full.shape[0]
    sim = lax.dot_general(tile, full, (((1,), (1,)), ((), ())),
                          preferred_element_type=jnp.float32)

    neg = jnp.float32(-3e38)
    lane_iota = lax.broadcasted_iota(jnp.int32, (_TILE, _LANES), 1)
    t1 = jnp.full((_TILE, _LANES), neg, jnp.float32)
    t2 = t1
    t3 = t1
    i1 = jnp.full((_TILE, _LANES), b, jnp.int32)
    i2 = i1
    i3 = i1
    for k in range(b // _LANES):
        v = sim[:, k * _LANES:(k + 1) * _LANES]
        iv = lane_iota + (k * _LANES)
        c1 = v > t1
        nt1 = jnp.maximum(t1, v)
        dv = jnp.minimum(t1, v)
        ni1 = jnp.where(c1, iv, i1)
        di = jnp.where(c1, i1, iv)
        c2 = dv > t2
        nt2 = jnp.maximum(t2, dv)
        dv2 = jnp.minimum(t2, dv)
        ni2 = jnp.where(c2, di, i2)
        di2 = jnp.where(c2, i2, di)
        c3 = dv2 > t3
        t3 = jnp.maximum(t3, dv2)
        i3 = jnp.where(c3, di2, i3)
        t1, t2, i1, i2 = nt1, nt2, ni1, ni2

    cand = jnp.concatenate([t1, t2, t3], axis=1)
    cidx = jnp.concatenate([i1, i2, i3], axis=1)
    vals, idxs = [], []
    for _ in range(_K):
        m = jnp.max(cand, axis=1)
        j = jnp.min(jnp.where(cand == m[:, None], cidx, b), axis=1)
        vals.append(m)
        idxs.append(j)
        cand = jnp.where(cidx == j[:, None], neg, cand)

    e0 = jnp.ones_like(vals[0])
    e1 = jnp.exp(vals[1] - vals[0])
    e2 = jnp.exp(vals[2] - vals[0])
    s = e0 + e1 + e2
    ws = [e0 / s, e1 / s, e2 / s]
    w_ref[...] = jnp.concatenate(
        [ws[0][:, None], ws[1][:, None], ws[2][:, None]], axis=1)
    nt = _TILE // _LANES
    wp_ref[...] = jnp.stack(
        [ws[0].reshape(nt, _LANES), ws[1].reshape(nt, _LANES),
         ws[2].reshape(nt, _LANES)] +
        [jnp.zeros((nt, _LANES), jnp.float32)] * 5, axis=1)
    idxp_ref[...] = jnp.stack(
        [idxs[0].reshape(nt, _LANES), idxs[1].reshape(nt, _LANES),
         idxs[2].reshape(nt, _LANES)] +
        [jnp.zeros((nt, _LANES), jnp.int32)] * 5, axis=1)


def _topk_tc(sess_emb, W1, b1, W2, b2, row0, nrows):
    b, d = sess_emb.shape
    grid = nrows // _TILE
    full_spec = lambda arr: pl.BlockSpec(arr.shape, lambda i: (0,) * arr.ndim)
    w_out = jax.ShapeDtypeStruct((nrows, _K), jnp.float32)
    wp_out = jax.ShapeDtypeStruct((nrows // 128, 8, 128), jnp.float32)
    idxp_out = jax.ShapeDtypeStruct((nrows // 128, 8, 128), jnp.int32)
    return pl.pallas_call(
        functools.partial(_topk_tc_body, row0=row0),
        grid=(grid,),
        in_specs=[full_spec(sess_emb), full_spec(W1), full_spec(b1),
                  full_spec(W2), full_spec(b2)],
        out_specs=[pl.BlockSpec((_TILE, _K), lambda i: (i, 0)),
                   pl.BlockSpec((_TILE // 128, 8, 128), lambda i: (i, 0, 0)),
                   pl.BlockSpec((_TILE // 128, 8, 128), lambda i: (i, 0, 0))],
        out_shape=[w_out, wp_out, idxp_out],
        scratch_shapes=[pltpu.VMEM((b, d), jnp.float32)],
    )(sess_emb, W1, b1, W2, b2)


def _gather_combine_sc(sess_emb, idx_p, w_p):
    b, d = sess_emb.shape
    nrows = idx_p.shape[1]
    info = plsc.get_sparse_core_info()
    nc, ns = info.num_cores, info.num_subcores
    nw = nc * ns
    rows_w = nrows // nw
    mesh = plsc.VectorSubcoreMesh(core_axis_name="c", subcore_axis_name="s")

    @functools.partial(
        pl.kernel,
        mesh=mesh,
        compiler_params=pltpu.CompilerParams(use_tc_tiling_on_sc=False),
        out_type=[jax.ShapeDtypeStruct((nrows, _K, d), jnp.float32),
                  jax.ShapeDtypeStruct((nrows, d), jnp.float32)],
        scratch_types=[
            [pltpu.VMEM((rows_w,), jnp.int32) for _ in range(_K)],
            [pltpu.VMEM((rows_w, d), jnp.float32) for _ in range(_K)],
            [pltpu.VMEM((rows_w + 16,), jnp.float32) for _ in range(_K)],
            pltpu.VMEM((rows_w, d), jnp.float32),
            pltpu.SemaphoreType.DMA,
        ],
    )
    def sc_kernel(emb_hbm, idx_hbm, w_hbm, topk_out, nb_out,
                  idx_vs, rows_vs, w_vs, acc_v, sem):
        wid = lax.axis_index("s") * nc + lax.axis_index("c")
        base_r = wid * rows_w
        tile_r = base_r // 128
        off_r = base_r % 128
        idx_cps = [pltpu.async_copy(
            idx_hbm.at[tile_r, k, pl.ds(off_r, rows_w)],
            idx_vs[k], sem) for k in range(_K)]
        w_cps = [pltpu.async_copy(
            w_hbm.at[tile_r, k, pl.ds(off_r, rows_w)],
            w_vs[k].at[pl.ds(0, rows_w)], sem)
                 for k in range(_K)]
        for c in idx_cps:
            c.wait()
        row_cps = [pltpu.async_copy(emb_hbm.at[idx_vs[k]], rows_vs[k], sem)
                   for k in range(_K)]
        for c in w_cps:
            c.wait()
        for c in row_cps:
            c.wait()
        out_cps = [pltpu.async_copy(rows_vs[k],
                                    topk_out.at[pl.ds(base_r, rows_w), k], sem)
                   for k in range(_K)]

        def body(r, carry):
            acc = (rows_vs[0][r, :] * w_vs[0][pl.ds(r, 16)][0]
                   + rows_vs[1][r, :] * w_vs[1][pl.ds(r, 16)][0]
                   + rows_vs[2][r, :] * w_vs[2][pl.ds(r, 16)][0])
            acc_v[r, :] = acc
            return carry

        lax.fori_loop(0, rows_w, body, 0)
        pltpu.sync_copy(acc_v, nb_out.at[pl.ds(base_r, rows_w)])
        for c in out_cps:
            c.wait()

    return sc_kernel(sess_emb, idx_p, w_p)


def kernel(sess_emb, pool_emb, W1, b1, W2, b2):
    del pool_emb
    b, d = sess_emb.shape
    half = b // 2
    b1r = b1.reshape(1, -1)
    b2r = b2.reshape(1, -1)
    w0, wp0, ip0 = _topk_tc(sess_emb, W1, b1r, W2, b2r, 0, half)
    st0, nb0 = _gather_combine_sc(sess_emb, ip0, wp0)
    w1, wp1, ip1 = _topk_tc(sess_emb, W1, b1r, W2, b2r, half, half)
    st1, nb1 = _gather_combine_sc(sess_emb, ip1, wp1)
    sess_topk = jnp.concatenate([st0, st1], axis=0)
    neighbor_sess = jnp.concatenate([nb0, nb1], axis=0)
    w = jnp.concatenate([w0, w1], axis=0)
    return (sess_topk, neighbor_sess, w)
